# baseline (speedup 1.0000x reference)
import jax
import jax.numpy as jnp
from jax.experimental import pallas as pl


def _copy_body(e_ref, o_ref):
    o_ref[...] = e_ref[...]


def kernel(emb_h, h, e, conv_w, conv_b, w2, b2, edge_index):
    E, H = e.shape
    block_rows = 2000
    if E % block_rows != 0:
        block_rows = E
    grid = (E // block_rows,)
    out = pl.pallas_call(
        _copy_body,
        grid=grid,
        in_specs=[pl.BlockSpec((block_rows, H), lambda i: (i, 0))],
        out_specs=pl.BlockSpec((block_rows, H), lambda i: (i, 0)),
        out_shape=jax.ShapeDtypeStruct((E, H), e.dtype),
    )(e)
    return out
